# baseline (speedup 1.0000x reference)
import jax
import jax.numpy as jnp
from jax import lax
from jax.experimental import pallas as pl
from jax.experimental.pallas import tpu as pltpu
from jax.experimental.pallas import tpu_sc as plsc

NC = 2
NS = 16
L = 16
NW = NC * NS

E = 320000
D = 128
EPW = E // NW
CH = 112
NFULL = EPW // CH
TAIL_OFF = EPW - CH
NCHUNK = NFULL + 1
SW = 129


def _body(x_hbm, src_hbm, dst_hbm, out_hbm,
          sidx_v, didx_v, out_v, rv0, rv1, rv2, tr_v,
          sem0, sem1, sem2):
  wid = lax.axis_index("s") * NC + lax.axis_index("c")
  base = wid * EPW
  rows0 = lax.broadcasted_iota(jnp.int32, (L,), 0)

  pltpu.sync_copy(src_hbm.at[pl.ds(base, EPW)], sidx_v)
  pltpu.sync_copy(dst_hbm.at[pl.ds(base, EPW)], didx_v)

  def fire(off, rv, sem):
    pltpu.async_copy(x_hbm.at[sidx_v.at[pl.ds(off, CH)]], rv.at[pl.ds(0, CH)], sem)
    pltpu.async_copy(x_hbm.at[didx_v.at[pl.ds(off, CH)]], rv.at[pl.ds(CH, CH)], sem)

  def wait(rv, sem):
    pltpu.make_async_copy(x_hbm.at[pl.ds(0, 2 * CH)], rv, sem).wait()

  colbase = rows0 * SW

  def compute(off, rv):
    def edge_body(e, carry):
      acc = jnp.zeros((L,), jnp.float32)
      for k in range(D // L):
        s = rv[e, pl.ds(k * L, L)]
        d = rv[CH + e, pl.ds(k * L, L)]
        acc = acc + s * d
      plsc.store_scatter(tr_v, [colbase + e], acc)
      return carry
    lax.fori_loop(0, CH, edge_body, 0, unroll=False)

    for cg in range(CH // L):
      tot = jnp.zeros((L,), jnp.float32)
      for k in range(L):
        tot = tot + plsc.load_gather(tr_v, [jnp.full((L,), k * SW + cg * L, jnp.int32) + rows0])
      out_v[pl.ds(off + cg * L, L)] = tot

  bufs = ((rv0, sem0), (rv1, sem1), (rv2, sem2))

  def off(c):
    return jnp.minimum(c * CH, TAIL_OFF)

  for i in range(3):
    fire(off(i), *bufs[i])

  def ring_body(t, carry):
    for i in range(3):
      c = 3 * t + i

      @pl.when(c < NCHUNK)
      def _do(i=i, c=c):
        wait(*bufs[i])
        compute(off(c), bufs[i][0])

        @pl.when(c + 3 < NCHUNK)
        def _refire():
          fire(off(c + 3), *bufs[i])
    return carry

  lax.fori_loop(0, (NCHUNK + 2) // 3, ring_body, 0, unroll=False)

  pltpu.sync_copy(out_v, out_hbm.at[pl.ds(base, EPW)])


@jax.jit
def kernel(x, edge_index):
  mesh = plsc.VectorSubcoreMesh(core_axis_name="c", subcore_axis_name="s")
  k = pl.kernel(
      _body,
      out_type=jax.ShapeDtypeStruct((E,), jnp.float32),
      mesh=mesh,
      compiler_params=pltpu.CompilerParams(needs_layout_passes=False),
      scratch_types=[
          pltpu.VMEM((EPW,), jnp.int32),
          pltpu.VMEM((EPW,), jnp.int32),
          pltpu.VMEM((EPW,), jnp.float32),
          pltpu.VMEM((2 * CH, D), jnp.float32),
          pltpu.VMEM((2 * CH, D), jnp.float32),
          pltpu.VMEM((2 * CH, D), jnp.float32),
          pltpu.VMEM((L * SW,), jnp.float32),
          pltpu.SemaphoreType.DMA,
          pltpu.SemaphoreType.DMA,
          pltpu.SemaphoreType.DMA,
      ],
  )
  return k(x, edge_index[0], edge_index[1])
